# baseline (speedup 1.0000x reference)
import functools

import jax
import jax.numpy as jnp
from jax import lax
from jax.experimental import pallas as pl
from jax.experimental.pallas import tpu as pltpu
from jax.experimental.pallas import tpu_sc as plsc

EMBED = 32
OUT = 64
B, L = 4096, 200
NTOK = B * L
PACK = 4
NPROW = NTOK // PACK

NC, NS = 2, 16
NW = NC * NS
TOK_PER_W = NTOK // NW
CHUNK = 128
NCHUNK = TOK_PER_W // CHUNK
LANES = 16

BLKP = 1600
NBLK = NPROW // BLKP
BPB = BLKP // L
RT = BLKP * PACK


def _sc_gather_body(idx_hbm, table_hbm, out_hbm, idx_p, buf0, buf1,
                    sem0, sem1):
    wid = lax.axis_index("s") * NC + lax.axis_index("c")
    row0 = wid * NCHUNK
    tok0 = wid * TOK_PER_W
    pltpu.sync_copy(idx_hbm.at[pl.ds(row0, NCHUNK)], idx_p)

    pltpu.async_copy(table_hbm.at[idx_p.at[0]], buf0, sem0)

    def body(i, _):
        j0 = 2 * i

        @pl.when(j0 + 1 < NCHUNK)
        def _():
            pltpu.async_copy(table_hbm.at[idx_p.at[j0 + 1]], buf1, sem1)

        pltpu.make_async_copy(table_hbm.at[idx_p.at[j0]], buf0, sem0).wait()
        pltpu.sync_copy(buf0, out_hbm.at[pl.ds(tok0 + j0 * CHUNK, CHUNK)])

        @pl.when(j0 + 2 < NCHUNK)
        def _():
            pltpu.async_copy(table_hbm.at[idx_p.at[j0 + 2]], buf0, sem0)

        @pl.when(j0 + 1 < NCHUNK)
        def _():
            pltpu.make_async_copy(table_hbm.at[idx_p.at[j0 + 1]], buf1, sem1).wait()
            pltpu.sync_copy(buf1, out_hbm.at[pl.ds(tok0 + (j0 + 1) * CHUNK, CHUNK)])

        return 0

    lax.fori_loop(0, (NCHUNK + 1) // 2, body, 0)


@functools.lru_cache(maxsize=1)
def _sc_gather():
    mesh = plsc.VectorSubcoreMesh(core_axis_name="c", subcore_axis_name="s")
    return functools.partial(
        pl.kernel,
        mesh=mesh,
        out_type=jax.ShapeDtypeStruct((NTOK, EMBED), jnp.float32),
        scratch_types=[
            pltpu.VMEM((NCHUNK, CHUNK), jnp.int32),
            pltpu.VMEM((CHUNK, EMBED), jnp.float32),
            pltpu.VMEM((CHUNK, EMBED), jnp.float32),
            pltpu.SemaphoreType.DMA,
            pltpu.SemaphoreType.DMA,
        ],
        compiler_params=pltpu.CompilerParams(use_tc_tiling_on_sc=False),
    )(_sc_gather_body)


def _p1_body(e_ref, w4_ref, b4_ref, s_ref):
    i = pl.program_id(0)
    e2 = e_ref[...].reshape(BLKP, PACK * EMBED)
    h2 = jnp.dot(e2, w4_ref[...], preferred_element_type=jnp.float32)
    ex = jnp.exp(jnp.tanh(h2 + b4_ref[...]))
    part = jnp.sum(ex.reshape(BPB, L, PACK * OUT), axis=0)

    @pl.when(i == 0)
    def _():
        s_ref[...] = part

    @pl.when(i != 0)
    def _():
        s_ref[...] = s_ref[...] + part


def _p2_body(e_ref, w4_ref, b4_ref, s_ref, o_ref):
    e2 = e_ref[...].reshape(BLKP, PACK * EMBED)
    h2 = jnp.dot(e2, w4_ref[...], preferred_element_type=jnp.float32)
    ex = jnp.exp(jnp.tanh(h2 + b4_ref[...]))
    s2 = s_ref[...]
    s = (lax.slice(s2, (0, 0), (L, OUT))
         + lax.slice(s2, (0, OUT), (L, 2 * OUT))
         + lax.slice(s2, (0, 2 * OUT), (L, 3 * OUT))
         + lax.slice(s2, (0, 3 * OUT), (L, 4 * OUT)))
    inv = 1.0 / s
    parts = []
    for j in range(PACK):
        pj = lax.slice(ex, (0, j * OUT), (BLKP, (j + 1) * OUT))
        parts.append((pj.reshape(BPB, L, OUT) * inv[None]).reshape(BLKP, OUT))
    o_ref[...] = jnp.concatenate(parts, axis=0).reshape(BPB * PACK, L, OUT)


_BELEM = BLKP * PACK * EMBED


def _pass1(emb1d, W4, b4):
    return pl.pallas_call(
        _p1_body,
        grid=(NBLK,),
        in_specs=[
            pl.BlockSpec((_BELEM,), lambda i: (i,)),
            pl.BlockSpec((PACK * EMBED, PACK * OUT), lambda i: (0, 0)),
            pl.BlockSpec((1, PACK * OUT), lambda i: (0, 0)),
        ],
        out_specs=pl.BlockSpec((L, PACK * OUT), lambda i: (0, 0)),
        out_shape=jax.ShapeDtypeStruct((L, PACK * OUT), jnp.float32),
    )(emb1d, W4, b4)


def _pass2(emb1d, W4, b4, s):
    return pl.pallas_call(
        _p2_body,
        grid=(NBLK,),
        in_specs=[
            pl.BlockSpec((_BELEM,), lambda i: (i,)),
            pl.BlockSpec((PACK * EMBED, PACK * OUT), lambda i: (0, 0)),
            pl.BlockSpec((1, PACK * OUT), lambda i: (0, 0)),
            pl.BlockSpec((L, PACK * OUT), lambda i: (0, 0)),
        ],
        out_specs=pl.BlockSpec((BPB * PACK, L, OUT), lambda i: (i, 0, 0)),
        out_shape=jax.ShapeDtypeStruct((B, L, OUT), jnp.float32),
    )(emb1d, W4, b4, s)


def kernel(x, table, W, b):
    idx = (x.reshape(NBLK, PACK, BLKP)
           .transpose(0, 2, 1)
           .reshape(NTOK // CHUNK, CHUNK)
           .astype(jnp.int32))
    emb = _sc_gather()(idx, table)
    emb1d = emb.reshape(NTOK * EMBED)
    W4 = jax.scipy.linalg.block_diag(W, W, W, W)
    b4 = jnp.tile(b, PACK).reshape(1, PACK * OUT)
    s = _pass1(emb1d, W4, b4)
    return _pass2(emb1d, W4, b4, s)


# speedup vs baseline: 1.1621x; 1.1621x over previous
import functools

import jax
import jax.numpy as jnp
from jax import lax
from jax.experimental import pallas as pl
from jax.experimental.pallas import tpu as pltpu
from jax.experimental.pallas import tpu_sc as plsc

EMBED = 32
OUT = 64
B, L = 4096, 200
NTOK = B * L
PACK = 4
NPROW = NTOK // PACK

NC, NS = 2, 16
NW = NC * NS
TOK_PER_W = NTOK // NW
CHUNK = 128
NCHUNK = TOK_PER_W // CHUNK
LANES = 16

BLKP = 3200
NBLK = NPROW // BLKP
BPB = BLKP // L
RT = BLKP * PACK


def _sc_gather_body(idx_hbm, table_hbm, out_hbm, idx_p, buf0, buf1,
                    sem0, sem1):
    wid = lax.axis_index("s") * NC + lax.axis_index("c")
    row0 = wid * NCHUNK
    tok0 = wid * TOK_PER_W
    pltpu.sync_copy(idx_hbm.at[pl.ds(row0, NCHUNK)], idx_p)

    pltpu.async_copy(table_hbm.at[idx_p.at[0]], buf0, sem0)

    def body(i, _):
        j0 = 2 * i

        @pl.when(j0 + 1 < NCHUNK)
        def _():
            pltpu.async_copy(table_hbm.at[idx_p.at[j0 + 1]], buf1, sem1)

        pltpu.make_async_copy(table_hbm.at[idx_p.at[j0]], buf0, sem0).wait()
        pltpu.sync_copy(buf0, out_hbm.at[pl.ds(tok0 + j0 * CHUNK, CHUNK)])

        @pl.when(j0 + 2 < NCHUNK)
        def _():
            pltpu.async_copy(table_hbm.at[idx_p.at[j0 + 2]], buf0, sem0)

        @pl.when(j0 + 1 < NCHUNK)
        def _():
            pltpu.make_async_copy(table_hbm.at[idx_p.at[j0 + 1]], buf1, sem1).wait()
            pltpu.sync_copy(buf1, out_hbm.at[pl.ds(tok0 + (j0 + 1) * CHUNK, CHUNK)])

        return 0

    lax.fori_loop(0, (NCHUNK + 1) // 2, body, 0)


@functools.lru_cache(maxsize=1)
def _sc_gather():
    mesh = plsc.VectorSubcoreMesh(core_axis_name="c", subcore_axis_name="s")
    return functools.partial(
        pl.kernel,
        mesh=mesh,
        out_type=jax.ShapeDtypeStruct((NTOK, EMBED), jnp.float32),
        scratch_types=[
            pltpu.VMEM((NCHUNK, CHUNK), jnp.int32),
            pltpu.VMEM((CHUNK, EMBED), jnp.float32),
            pltpu.VMEM((CHUNK, EMBED), jnp.float32),
            pltpu.SemaphoreType.DMA,
            pltpu.SemaphoreType.DMA,
        ],
        compiler_params=pltpu.CompilerParams(use_tc_tiling_on_sc=False),
    )(_sc_gather_body)


def _p1_body(e_ref, w4_ref, b4_ref, s_ref):
    i = pl.program_id(0)
    e2 = e_ref[...].reshape(BLKP, PACK * EMBED)
    h2 = jnp.dot(e2, w4_ref[...], preferred_element_type=jnp.float32)
    ex = jnp.exp(jnp.tanh(h2 + b4_ref[...]))
    part = jnp.sum(ex.reshape(BPB, L, PACK * OUT), axis=0)

    @pl.when(i == 0)
    def _():
        s_ref[...] = part

    @pl.when(i != 0)
    def _():
        s_ref[...] = s_ref[...] + part


def _p2_body(e_ref, w4_ref, b4_ref, s_ref, o_ref):
    e2 = e_ref[...].reshape(BLKP, PACK * EMBED)
    h2 = jnp.dot(e2, w4_ref[...], preferred_element_type=jnp.float32)
    ex = jnp.exp(jnp.tanh(h2 + b4_ref[...]))
    s2 = s_ref[...]
    s = (lax.slice(s2, (0, 0), (L, OUT))
         + lax.slice(s2, (0, OUT), (L, 2 * OUT))
         + lax.slice(s2, (0, 2 * OUT), (L, 3 * OUT))
         + lax.slice(s2, (0, 3 * OUT), (L, 4 * OUT)))
    inv = 1.0 / s
    parts = []
    for j in range(PACK):
        pj = lax.slice(ex, (0, j * OUT), (BLKP, (j + 1) * OUT))
        parts.append((pj.reshape(BPB, L, OUT) * inv[None]).reshape(BLKP, OUT))
    o_ref[...] = jnp.concatenate(parts, axis=0)


_BELEM = BLKP * PACK * EMBED


def _pass1(emb1d, W4, b4):
    return pl.pallas_call(
        _p1_body,
        grid=(NBLK,),
        in_specs=[
            pl.BlockSpec((_BELEM,), lambda i: (i,)),
            pl.BlockSpec((PACK * EMBED, PACK * OUT), lambda i: (0, 0)),
            pl.BlockSpec((1, PACK * OUT), lambda i: (0, 0)),
        ],
        out_specs=pl.BlockSpec((L, PACK * OUT), lambda i: (0, 0)),
        out_shape=jax.ShapeDtypeStruct((L, PACK * OUT), jnp.float32),
    )(emb1d, W4, b4)


def _pass2(emb1d, W4, b4, s):
    return pl.pallas_call(
        _p2_body,
        grid=(NBLK,),
        in_specs=[
            pl.BlockSpec((_BELEM,), lambda i: (i,)),
            pl.BlockSpec((PACK * EMBED, PACK * OUT), lambda i: (0, 0)),
            pl.BlockSpec((1, PACK * OUT), lambda i: (0, 0)),
            pl.BlockSpec((L, PACK * OUT), lambda i: (0, 0)),
        ],
        out_specs=pl.BlockSpec((RT, OUT), lambda i: (i, 0)),
        out_shape=jax.ShapeDtypeStruct((NTOK, OUT), jnp.float32),
    )(emb1d, W4, b4, s)


def kernel(x, table, W, b):
    idx = (x.reshape(NBLK, PACK, BLKP)
           .transpose(0, 2, 1)
           .reshape(NTOK // CHUNK, CHUNK)
           .astype(jnp.int32))
    emb = _sc_gather()(idx, table)
    emb1d = emb.reshape(NTOK * EMBED)
    W4 = jax.scipy.linalg.block_diag(W, W, W, W)
    b4 = jnp.tile(b, PACK).reshape(1, PACK * OUT)
    s = _pass1(emb1d, W4, b4)
    out = _pass2(emb1d, W4, b4, s)
    return out.reshape(B, L, OUT)


# speedup vs baseline: 1.1834x; 1.0184x over previous
import functools

import jax
import jax.numpy as jnp
from jax import lax
from jax.experimental import pallas as pl
from jax.experimental.pallas import tpu as pltpu
from jax.experimental.pallas import tpu_sc as plsc

EMBED = 32
OUT = 64
B, L = 4096, 200
NTOK = B * L
PACK = 4
NPROW = NTOK // PACK

NC, NS = 2, 16
NW = NC * NS
TOK_PER_W = NTOK // NW
CHUNK = 128
NCHUNK = TOK_PER_W // CHUNK
LANES = 16

BLKP = 6400
NBLK = NPROW // BLKP
BPB = BLKP // L
RT = BLKP * PACK


def _sc_gather_body(idx_hbm, table_hbm, out_hbm, idx_p, buf0, buf1,
                    sem0, sem1):
    wid = lax.axis_index("s") * NC + lax.axis_index("c")
    row0 = wid * NCHUNK
    tok0 = wid * TOK_PER_W
    pltpu.sync_copy(idx_hbm.at[pl.ds(row0, NCHUNK)], idx_p)

    pltpu.async_copy(table_hbm.at[idx_p.at[0]], buf0, sem0)

    def body(i, _):
        j0 = 2 * i

        @pl.when(j0 + 1 < NCHUNK)
        def _():
            pltpu.async_copy(table_hbm.at[idx_p.at[j0 + 1]], buf1, sem1)

        pltpu.make_async_copy(table_hbm.at[idx_p.at[j0]], buf0, sem0).wait()
        pltpu.sync_copy(buf0, out_hbm.at[pl.ds(tok0 + j0 * CHUNK, CHUNK)])

        @pl.when(j0 + 2 < NCHUNK)
        def _():
            pltpu.async_copy(table_hbm.at[idx_p.at[j0 + 2]], buf0, sem0)

        @pl.when(j0 + 1 < NCHUNK)
        def _():
            pltpu.make_async_copy(table_hbm.at[idx_p.at[j0 + 1]], buf1, sem1).wait()
            pltpu.sync_copy(buf1, out_hbm.at[pl.ds(tok0 + (j0 + 1) * CHUNK, CHUNK)])

        return 0

    lax.fori_loop(0, (NCHUNK + 1) // 2, body, 0)


@functools.lru_cache(maxsize=1)
def _sc_gather():
    mesh = plsc.VectorSubcoreMesh(core_axis_name="c", subcore_axis_name="s")
    return functools.partial(
        pl.kernel,
        mesh=mesh,
        out_type=jax.ShapeDtypeStruct((NTOK, EMBED), jnp.float32),
        scratch_types=[
            pltpu.VMEM((NCHUNK, CHUNK), jnp.int32),
            pltpu.VMEM((CHUNK, EMBED), jnp.float32),
            pltpu.VMEM((CHUNK, EMBED), jnp.float32),
            pltpu.SemaphoreType.DMA,
            pltpu.SemaphoreType.DMA,
        ],
        compiler_params=pltpu.CompilerParams(use_tc_tiling_on_sc=False),
    )(_sc_gather_body)


def _p1_body(e_ref, w4_ref, b4_ref, s_ref):
    i = pl.program_id(0)
    e2 = e_ref[...].reshape(BLKP, PACK * EMBED)
    h2 = jnp.dot(e2, w4_ref[...], preferred_element_type=jnp.float32)
    ex = jnp.exp(jnp.tanh(h2 + b4_ref[...]))
    part = jnp.sum(ex.reshape(BPB, L, PACK * OUT), axis=0)

    @pl.when(i == 0)
    def _():
        s_ref[...] = part

    @pl.when(i != 0)
    def _():
        s_ref[...] = s_ref[...] + part


def _p2_body(e_ref, w4_ref, b4_ref, s_ref, o_ref):
    e2 = e_ref[...].reshape(BLKP, PACK * EMBED)
    h2 = jnp.dot(e2, w4_ref[...], preferred_element_type=jnp.float32)
    ex = jnp.exp(jnp.tanh(h2 + b4_ref[...]))
    s2 = s_ref[...]
    s = (lax.slice(s2, (0, 0), (L, OUT))
         + lax.slice(s2, (0, OUT), (L, 2 * OUT))
         + lax.slice(s2, (0, 2 * OUT), (L, 3 * OUT))
         + lax.slice(s2, (0, 3 * OUT), (L, 4 * OUT)))
    inv = 1.0 / s
    parts = []
    for j in range(PACK):
        pj = lax.slice(ex, (0, j * OUT), (BLKP, (j + 1) * OUT))
        parts.append((pj.reshape(BPB, L, OUT) * inv[None]).reshape(BLKP, OUT))
    o_ref[...] = jnp.concatenate(parts, axis=0)


_BELEM = BLKP * PACK * EMBED


def _pass1(emb1d, W4, b4):
    return pl.pallas_call(
        _p1_body,
        grid=(NBLK,),
        in_specs=[
            pl.BlockSpec((_BELEM,), lambda i: (i,)),
            pl.BlockSpec((PACK * EMBED, PACK * OUT), lambda i: (0, 0)),
            pl.BlockSpec((1, PACK * OUT), lambda i: (0, 0)),
        ],
        out_specs=pl.BlockSpec((L, PACK * OUT), lambda i: (0, 0)),
        out_shape=jax.ShapeDtypeStruct((L, PACK * OUT), jnp.float32),
    )(emb1d, W4, b4)


def _pass2(emb1d, W4, b4, s):
    return pl.pallas_call(
        _p2_body,
        grid=(NBLK,),
        in_specs=[
            pl.BlockSpec((_BELEM,), lambda i: (i,)),
            pl.BlockSpec((PACK * EMBED, PACK * OUT), lambda i: (0, 0)),
            pl.BlockSpec((1, PACK * OUT), lambda i: (0, 0)),
            pl.BlockSpec((L, PACK * OUT), lambda i: (0, 0)),
        ],
        out_specs=pl.BlockSpec((RT, OUT), lambda i: (i, 0)),
        out_shape=jax.ShapeDtypeStruct((NTOK, OUT), jnp.float32),
    )(emb1d, W4, b4, s)


def kernel(x, table, W, b):
    idx = (x.reshape(NBLK, PACK, BLKP)
           .transpose(0, 2, 1)
           .reshape(NTOK // CHUNK, CHUNK)
           .astype(jnp.int32))
    emb = _sc_gather()(idx, table)
    emb1d = emb.reshape(NTOK * EMBED)
    W4 = jax.scipy.linalg.block_diag(W, W, W, W)
    b4 = jnp.tile(b, PACK).reshape(1, PACK * OUT)
    s = _pass1(emb1d, W4, b4)
    out = _pass2(emb1d, W4, b4, s)
    return out.reshape(B, L, OUT)


# speedup vs baseline: 1.1841x; 1.0006x over previous
import functools

import jax
import jax.numpy as jnp
from jax import lax
from jax.experimental import pallas as pl
from jax.experimental.pallas import tpu as pltpu
from jax.experimental.pallas import tpu_sc as plsc

EMBED = 32
OUT = 64
B, L = 4096, 200
NTOK = B * L
PACK = 4
NPROW = NTOK // PACK

NC, NS = 2, 16
NW = NC * NS
TOK_PER_W = NTOK // NW
CHUNK = 128
NCHUNK = TOK_PER_W // CHUNK
LANES = 16

BLKP = 6400
NBLK = NPROW // BLKP
BPB = BLKP // L
RT = BLKP * PACK


def _sc_gather_body(idx_hbm, table_hbm, out_hbm, idx_p, buf0, buf1,
                    sem0, sem1):
    wid = lax.axis_index("s") * NC + lax.axis_index("c")
    row0 = wid * NCHUNK
    tok0 = wid * TOK_PER_W
    pltpu.sync_copy(idx_hbm.at[pl.ds(row0, NCHUNK)], idx_p)

    pltpu.async_copy(table_hbm.at[idx_p.at[0]], buf0, sem0)

    def body(i, _):
        j0 = 2 * i

        @pl.when(j0 + 1 < NCHUNK)
        def _():
            pltpu.async_copy(table_hbm.at[idx_p.at[j0 + 1]], buf1, sem1)

        pltpu.make_async_copy(table_hbm.at[idx_p.at[j0]], buf0, sem0).wait()
        pltpu.sync_copy(buf0, out_hbm.at[pl.ds(tok0 + j0 * CHUNK, CHUNK)])

        @pl.when(j0 + 2 < NCHUNK)
        def _():
            pltpu.async_copy(table_hbm.at[idx_p.at[j0 + 2]], buf0, sem0)

        @pl.when(j0 + 1 < NCHUNK)
        def _():
            pltpu.make_async_copy(table_hbm.at[idx_p.at[j0 + 1]], buf1, sem1).wait()
            pltpu.sync_copy(buf1, out_hbm.at[pl.ds(tok0 + (j0 + 1) * CHUNK, CHUNK)])

        return 0

    lax.fori_loop(0, (NCHUNK + 1) // 2, body, 0)


@functools.lru_cache(maxsize=1)
def _sc_gather():
    mesh = plsc.VectorSubcoreMesh(core_axis_name="c", subcore_axis_name="s")
    return functools.partial(
        pl.kernel,
        mesh=mesh,
        out_type=jax.ShapeDtypeStruct((NTOK, EMBED), jnp.float32),
        scratch_types=[
            pltpu.VMEM((NCHUNK, CHUNK), jnp.int32),
            pltpu.VMEM((CHUNK, EMBED), jnp.float32),
            pltpu.VMEM((CHUNK, EMBED), jnp.float32),
            pltpu.SemaphoreType.DMA,
            pltpu.SemaphoreType.DMA,
        ],
        compiler_params=pltpu.CompilerParams(use_tc_tiling_on_sc=False),
    )(_sc_gather_body)


def _p1_body(e_ref, w4_ref, b4_ref, s_ref):
    i = pl.program_id(0)
    e2 = e_ref[...]
    h2 = jnp.dot(e2, w4_ref[...], preferred_element_type=jnp.float32)
    ex = jnp.exp(jnp.tanh(h2 + b4_ref[...]))
    part = jnp.sum(ex.reshape(BPB, L, PACK * OUT), axis=0)

    @pl.when(i == 0)
    def _():
        s_ref[...] = part

    @pl.when(i != 0)
    def _():
        s_ref[...] = s_ref[...] + part


def _p2_body(e_ref, w4_ref, b4_ref, s_ref, o_ref):
    e2 = e_ref[...]
    h2 = jnp.dot(e2, w4_ref[...], preferred_element_type=jnp.float32)
    ex = jnp.exp(jnp.tanh(h2 + b4_ref[...]))
    s2 = s_ref[...]
    s = (lax.slice(s2, (0, 0), (L, OUT))
         + lax.slice(s2, (0, OUT), (L, 2 * OUT))
         + lax.slice(s2, (0, 2 * OUT), (L, 3 * OUT))
         + lax.slice(s2, (0, 3 * OUT), (L, 4 * OUT)))
    inv = 1.0 / s
    parts = []
    for j in range(PACK):
        pj = lax.slice(ex, (0, j * OUT), (BLKP, (j + 1) * OUT))
        parts.append((pj.reshape(BPB, L, OUT) * inv[None]).reshape(BLKP, OUT))
    o_ref[...] = jnp.concatenate(parts, axis=0)


_BELEM = BLKP * PACK * EMBED


def _pass1(emb1d, W4, b4):
    return pl.pallas_call(
        _p1_body,
        grid=(NBLK,),
        in_specs=[
            pl.BlockSpec((BLKP, PACK * EMBED), lambda i: (i, 0)),
            pl.BlockSpec((PACK * EMBED, PACK * OUT), lambda i: (0, 0)),
            pl.BlockSpec((1, PACK * OUT), lambda i: (0, 0)),
        ],
        out_specs=pl.BlockSpec((L, PACK * OUT), lambda i: (0, 0)),
        out_shape=jax.ShapeDtypeStruct((L, PACK * OUT), jnp.float32),
    )(emb1d, W4, b4)


def _pass2(emb1d, W4, b4, s):
    return pl.pallas_call(
        _p2_body,
        grid=(NBLK,),
        in_specs=[
            pl.BlockSpec((BLKP, PACK * EMBED), lambda i: (i, 0)),
            pl.BlockSpec((PACK * EMBED, PACK * OUT), lambda i: (0, 0)),
            pl.BlockSpec((1, PACK * OUT), lambda i: (0, 0)),
            pl.BlockSpec((L, PACK * OUT), lambda i: (0, 0)),
        ],
        out_specs=pl.BlockSpec((RT, OUT), lambda i: (i, 0)),
        out_shape=jax.ShapeDtypeStruct((NTOK, OUT), jnp.float32),
    )(emb1d, W4, b4, s)


def kernel(x, table, W, b):
    idx = (x.reshape(NBLK, PACK, BLKP)
           .transpose(0, 2, 1)
           .reshape(NTOK // CHUNK, CHUNK)
           .astype(jnp.int32))
    emb2 = _sc_gather()(idx, table).reshape(NPROW, PACK * EMBED)
    W4 = jax.scipy.linalg.block_diag(W, W, W, W)
    b4 = jnp.tile(b, PACK).reshape(1, PACK * OUT)
    s = _pass1(emb2, W4, b4)
    out = _pass2(emb2, W4, b4, s)
    return out.reshape(B, L, OUT)


# speedup vs baseline: 1.2074x; 1.0196x over previous
import functools

import jax
import jax.numpy as jnp
from jax import lax
from jax.experimental import pallas as pl
from jax.experimental.pallas import tpu as pltpu
from jax.experimental.pallas import tpu_sc as plsc

EMBED = 32
OUT = 64
B, L = 4096, 200
NTOK = B * L
PACK = 4
NPROW = NTOK // PACK

NC, NS = 2, 16
NW = NC * NS
TOK_PER_W = NTOK // NW
CHUNK = 128
NCHUNK = TOK_PER_W // CHUNK
LANES = 16

BLKP = 6400
NBLK = NPROW // BLKP
BPB = BLKP // L
RT = BLKP * PACK


def _sc_gather_body(idx_hbm, table_hbm, out_hbm, idx_p, buf0, buf1, buf2, buf3,
                    semg, semw):
    wid = lax.axis_index("s") * NC + lax.axis_index("c")
    row0 = wid * NCHUNK
    tok0 = wid * TOK_PER_W
    bufs = (buf0, buf1, buf2, buf3)
    pltpu.sync_copy(idx_hbm.at[pl.ds(row0, NCHUNK)], idx_p)

    pltpu.async_copy(table_hbm.at[idx_p.at[0]], buf0, semg)
    pltpu.async_copy(table_hbm.at[idx_p.at[1]], buf1, semg)

    def body(i, _):
        for bsel in range(4):
            j = 4 * i + bsel
            buf = bufs[bsel]
            pltpu.make_async_copy(table_hbm.at[idx_p.at[j]], buf, semg).wait()
            pltpu.async_copy(buf, out_hbm.at[pl.ds(tok0 + j * CHUNK, CHUNK)],
                             semw)

            @pl.when(j >= 2)
            def _():
                pltpu.make_async_copy(
                    bufs[(bsel + 2) % 4],
                    out_hbm.at[pl.ds(tok0 + (j - 2) * CHUNK, CHUNK)],
                    semw).wait()

            @pl.when(j + 2 < NCHUNK)
            def _():
                pltpu.async_copy(table_hbm.at[idx_p.at[j + 2]],
                                 bufs[(bsel + 2) % 4], semg)
        return 0

    lax.fori_loop(0, NCHUNK // 4, body, 0)
    pltpu.make_async_copy(
        buf2, out_hbm.at[pl.ds(tok0 + (NCHUNK - 2) * CHUNK, CHUNK)], semw).wait()
    pltpu.make_async_copy(
        buf3, out_hbm.at[pl.ds(tok0 + (NCHUNK - 1) * CHUNK, CHUNK)], semw).wait()


@functools.lru_cache(maxsize=1)
def _sc_gather():
    mesh = plsc.VectorSubcoreMesh(core_axis_name="c", subcore_axis_name="s")
    return functools.partial(
        pl.kernel,
        mesh=mesh,
        out_type=jax.ShapeDtypeStruct((NTOK, EMBED), jnp.float32),
        scratch_types=[
            pltpu.VMEM((NCHUNK, CHUNK), jnp.int32),
            pltpu.VMEM((CHUNK, EMBED), jnp.float32),
            pltpu.VMEM((CHUNK, EMBED), jnp.float32),
            pltpu.VMEM((CHUNK, EMBED), jnp.float32),
            pltpu.VMEM((CHUNK, EMBED), jnp.float32),
            pltpu.SemaphoreType.DMA,
            pltpu.SemaphoreType.DMA,
        ],
        compiler_params=pltpu.CompilerParams(use_tc_tiling_on_sc=False),
    )(_sc_gather_body)


def _p1_body(e_ref, w4_ref, b4_ref, s_ref):
    i = pl.program_id(0)
    e2 = e_ref[...]
    h2 = jnp.dot(e2, w4_ref[...], preferred_element_type=jnp.float32)
    ex = jnp.exp(jnp.tanh(h2 + b4_ref[...]))
    part = jnp.sum(ex.reshape(BPB, L, PACK * OUT), axis=0)

    @pl.when(i == 0)
    def _():
        s_ref[...] = part

    @pl.when(i != 0)
    def _():
        s_ref[...] = s_ref[...] + part


def _p2_body(e_ref, w4_ref, b4_ref, s_ref, o_ref):
    e2 = e_ref[...]
    h2 = jnp.dot(e2, w4_ref[...], preferred_element_type=jnp.float32)
    ex = jnp.exp(jnp.tanh(h2 + b4_ref[...]))
    s2 = s_ref[...]
    s = (lax.slice(s2, (0, 0), (L, OUT))
         + lax.slice(s2, (0, OUT), (L, 2 * OUT))
         + lax.slice(s2, (0, 2 * OUT), (L, 3 * OUT))
         + lax.slice(s2, (0, 3 * OUT), (L, 4 * OUT)))
    inv = 1.0 / s
    parts = []
    for j in range(PACK):
        pj = lax.slice(ex, (0, j * OUT), (BLKP, (j + 1) * OUT))
        parts.append((pj.reshape(BPB, L, OUT) * inv[None]).reshape(BLKP, OUT))
    o_ref[...] = jnp.concatenate(parts, axis=0)


_BELEM = BLKP * PACK * EMBED


def _pass1(emb1d, W4, b4):
    return pl.pallas_call(
        _p1_body,
        grid=(NBLK,),
        in_specs=[
            pl.BlockSpec((BLKP, PACK * EMBED), lambda i: (i, 0)),
            pl.BlockSpec((PACK * EMBED, PACK * OUT), lambda i: (0, 0)),
            pl.BlockSpec((1, PACK * OUT), lambda i: (0, 0)),
        ],
        out_specs=pl.BlockSpec((L, PACK * OUT), lambda i: (0, 0)),
        out_shape=jax.ShapeDtypeStruct((L, PACK * OUT), jnp.float32),
    )(emb1d, W4, b4)


def _pass2(emb1d, W4, b4, s):
    return pl.pallas_call(
        _p2_body,
        grid=(NBLK,),
        in_specs=[
            pl.BlockSpec((BLKP, PACK * EMBED), lambda i: (i, 0)),
            pl.BlockSpec((PACK * EMBED, PACK * OUT), lambda i: (0, 0)),
            pl.BlockSpec((1, PACK * OUT), lambda i: (0, 0)),
            pl.BlockSpec((L, PACK * OUT), lambda i: (0, 0)),
        ],
        out_specs=pl.BlockSpec((RT, OUT), lambda i: (i, 0)),
        out_shape=jax.ShapeDtypeStruct((NTOK, OUT), jnp.float32),
    )(emb1d, W4, b4, s)


def kernel(x, table, W, b):
    idx = (x.reshape(NBLK, PACK, BLKP)
           .transpose(0, 2, 1)
           .reshape(NTOK // CHUNK, CHUNK)
           .astype(jnp.int32))
    emb2 = _sc_gather()(idx, table).reshape(NPROW, PACK * EMBED)
    W4 = jax.scipy.linalg.block_diag(W, W, W, W)
    b4 = jnp.tile(b, PACK).reshape(1, PACK * OUT)
    s = _pass1(emb2, W4, b4)
    out = _pass2(emb2, W4, b4, s)
    return out.reshape(B, L, OUT)


# speedup vs baseline: 1.2494x; 1.0348x over previous
import functools

import jax
import jax.numpy as jnp
import jax.scipy.linalg
from jax import lax
from jax.experimental import pallas as pl
from jax.experimental.pallas import tpu as pltpu
from jax.experimental.pallas import tpu_sc as plsc

EMBED = 32
OUT = 64
B, L = 4096, 200
NTOK = B * L
PACK = 4
NPROW = NTOK // PACK

NC, NS = 2, 16
NW = NC * NS
TOK_PER_W = NTOK // NW
CHUNK = 128
NCHUNK = TOK_PER_W // CHUNK

BLKP = 6400
NBLK = NPROW // BLKP
BPB = BLKP // L
RT = BLKP * PACK


NBUF = 20
DEPTH = NBUF // 2


def _sc_gather_body(idx_hbm, table_hbm, out_hbm, idx_p, *rest):
    bufs = rest[:NBUF]
    semg, semw = rest[NBUF], rest[NBUF + 1]
    wid = lax.axis_index("s") * NC + lax.axis_index("c")
    row0 = wid * NCHUNK
    tok0 = wid * TOK_PER_W
    pltpu.sync_copy(idx_hbm.at[pl.ds(row0, NCHUNK)], idx_p)

    for k in range(DEPTH):
        pltpu.async_copy(table_hbm.at[idx_p.at[k]], bufs[k], semg)

    def body(i, _):
        for bsel in range(NBUF):
            j = NBUF * i + bsel
            buf = bufs[bsel]
            pltpu.make_async_copy(table_hbm.at[idx_p.at[j]], buf, semg).wait()
            pltpu.async_copy(buf, out_hbm.at[pl.ds(tok0 + j * CHUNK, CHUNK)],
                             semw)

            @pl.when(j >= DEPTH)
            def _():
                pltpu.make_async_copy(
                    bufs[(bsel + DEPTH) % NBUF],
                    out_hbm.at[pl.ds(tok0 + (j - DEPTH) * CHUNK, CHUNK)],
                    semw).wait()

            @pl.when(j + DEPTH < NCHUNK)
            def _():
                pltpu.async_copy(table_hbm.at[idx_p.at[j + DEPTH]],
                                 bufs[(bsel + DEPTH) % NBUF], semg)
        return 0

    lax.fori_loop(0, NCHUNK // NBUF, body, 0)
    for k in range(DEPTH):
        pltpu.make_async_copy(
            bufs[DEPTH + k],
            out_hbm.at[pl.ds(tok0 + (NCHUNK - DEPTH + k) * CHUNK, CHUNK)],
            semw).wait()


@functools.lru_cache(maxsize=1)
def _sc_gather():
    mesh = plsc.VectorSubcoreMesh(core_axis_name="c", subcore_axis_name="s")
    return functools.partial(
        pl.kernel,
        mesh=mesh,
        out_type=jax.ShapeDtypeStruct((NTOK, EMBED), jnp.float32),
        scratch_types=[
            pltpu.VMEM((NCHUNK, CHUNK), jnp.int32),
        ] + [pltpu.VMEM((CHUNK, EMBED), jnp.float32)] * NBUF + [
            pltpu.SemaphoreType.DMA,
            pltpu.SemaphoreType.DMA,
        ],
        compiler_params=pltpu.CompilerParams(use_tc_tiling_on_sc=False),
    )(_sc_gather_body)


P1BLK = 12800
P1NBLK = NPROW // P1BLK
P1BPB = P1BLK // L


def _p1_body(e_ref, w4_ref, b4_ref, s_ref):
    i = pl.program_id(0)
    e2 = e_ref[...]
    h2 = jnp.dot(e2, w4_ref[...], preferred_element_type=jnp.float32)
    ex = jnp.exp(jnp.tanh(h2 + b4_ref[...]))
    part = jnp.sum(ex.reshape(P1BPB, L, PACK * OUT), axis=0)

    @pl.when(i == 0)
    def _():
        s_ref[...] = part

    @pl.when(i != 0)
    def _():
        s_ref[...] = s_ref[...] + part


def _p2_body(e_ref, w4_ref, b4_ref, s_ref, o_ref):
    e2 = e_ref[...]
    h2 = jnp.dot(e2, w4_ref[...], preferred_element_type=jnp.float32)
    ex = jnp.exp(jnp.tanh(h2 + b4_ref[...]))
    s2 = s_ref[...]
    s = (lax.slice(s2, (0, 0), (L, OUT))
         + lax.slice(s2, (0, OUT), (L, 2 * OUT))
         + lax.slice(s2, (0, 2 * OUT), (L, 3 * OUT))
         + lax.slice(s2, (0, 3 * OUT), (L, 4 * OUT)))
    inv = 1.0 / s
    parts = []
    for j in range(PACK):
        pj = lax.slice(ex, (0, j * OUT), (BLKP, (j + 1) * OUT))
        parts.append((pj.reshape(BPB, L, OUT) * inv[None]).reshape(BLKP, OUT))
    o_ref[...] = jnp.concatenate(parts, axis=0)


def _pass1(emb2, W4, b4):
    return pl.pallas_call(
        _p1_body,
        grid=(P1NBLK,),
        in_specs=[
            pl.BlockSpec((P1BLK, PACK * EMBED), lambda i: (i, 0)),
            pl.BlockSpec((PACK * EMBED, PACK * OUT), lambda i: (0, 0)),
            pl.BlockSpec((1, PACK * OUT), lambda i: (0, 0)),
        ],
        out_specs=pl.BlockSpec((L, PACK * OUT), lambda i: (0, 0)),
        out_shape=jax.ShapeDtypeStruct((L, PACK * OUT), jnp.float32),
    )(emb2, W4, b4)


def _pass2(emb2, W4, b4, s):
    return pl.pallas_call(
        _p2_body,
        grid=(NBLK,),
        in_specs=[
            pl.BlockSpec((BLKP, PACK * EMBED), lambda i: (i, 0)),
            pl.BlockSpec((PACK * EMBED, PACK * OUT), lambda i: (0, 0)),
            pl.BlockSpec((1, PACK * OUT), lambda i: (0, 0)),
            pl.BlockSpec((L, PACK * OUT), lambda i: (0, 0)),
        ],
        out_specs=pl.BlockSpec((RT, OUT), lambda i: (i, 0)),
        out_shape=jax.ShapeDtypeStruct((NTOK, OUT), jnp.float32),
    )(emb2, W4, b4, s)


def kernel(x, table, W, b):
    idx = (x.reshape(NBLK, PACK, BLKP)
           .transpose(0, 2, 1)
           .reshape(NTOK // CHUNK, CHUNK)
           .astype(jnp.int32))
    emb2 = _sc_gather()(idx, table).reshape(NPROW, PACK * EMBED)
    W4 = jax.scipy.linalg.block_diag(W, W, W, W)
    b4 = jnp.tile(b, PACK).reshape(1, PACK * OUT)
    s = _pass1(emb2, W4, b4)
    out = _pass2(emb2, W4, b4, s)
    return out.reshape(B, L, OUT)
